# baseline (speedup 1.0000x reference)
import functools

import jax
import jax.numpy as jnp
from jax import lax
from jax.experimental import pallas as pl
from jax.experimental.pallas import tpu as pltpu
from jax.experimental.pallas import tpu_sc as plsc

N = 10000
E = 320000
D = 128
DOUT = 64
G = 64
DEGW = 16
NC = 2
NS = 16
NW = NC * NS
CH = 64
GRP = 6
EPW = E // NW
CPW = 158
EW = CPW * CH
NPADE = EW - EPW
NGRP = CPW // GRP
TAIL = CPW - NGRP * GRP
NTRASH = N + 8
BN = 1000

RFULL = 632
_CHUNKS_HEAD = [(0, 128), (128, 128), (256, 128), (384, 128)]

_MESH = dict(core_axis_name="c", subcore_axis_name="s")


def _foreach_tile_rows(s, copy_fn):
    cps = []
    base = s * RFULL
    for k, (off, ln) in enumerate(_CHUNKS_HEAD):
        cps.append(copy_fn(pl.multiple_of(base + off, 8), ln, k))

    @pl.when(s < NS - 1)
    def _():
        copy_fn(pl.multiple_of(base + 512, 8), RFULL - 512, 4).wait()

    @pl.when(s == NS - 1)
    def _():
        copy_fn(pl.multiple_of(base + 512, 8), N - (NS - 1) * RFULL - 512, 4).wait()

    for cp in cps:
        cp.wait()


@functools.partial(
    pl.kernel,
    mesh=plsc.VectorSubcoreMesh(**_MESH),
    out_type=jax.ShapeDtypeStruct((NC, N, D), jnp.float32),
    scratch_types=[
        [pltpu.VMEM((2, CH), jnp.int32)] * GRP,
        [pltpu.VMEM((CH, D), jnp.float32)] * GRP,
        pltpu.VMEM_SHARED((NTRASH, D), jnp.float32),
        [pltpu.SemaphoreType.DMA] * GRP,
        [pltpu.SemaphoreType.DMA] * GRP,
        [pltpu.SemaphoreType.DMA] * GRP,
    ],
)
def _sc_spmm(v_hbm, idx_hbm, zeros_hbm, out_hbm,
             idxp, rows, acc, isems, gsems, ssems):
    c = lax.axis_index("c")
    s = lax.axis_index("s")
    wid = s * NC + c

    _foreach_tile_rows(s, lambda off, ln, k: pltpu.async_copy(
        zeros_hbm.at[pl.ds(0, ln)], acc.at[pl.ds(off, ln)], gsems[k]))

    @pl.when(s == 0)
    def _():
        pltpu.sync_copy(zeros_hbm.at[pl.ds(0, NTRASH - N)], acc.at[pl.ds(N, NTRASH - N)])

    plsc.subcore_barrier()

    def wait_idx(b):
        pltpu.make_async_copy(idx_hbm.at[wid, 0], idxp[b], isems[b]).wait()

    def wait_gather(b):
        pltpu.make_async_copy(v_hbm.at[idxp[b].at[0]], rows[b], gsems[b]).wait()

    def wait_scatter(b):
        pltpu.make_async_copy(rows[b], acc.at[idxp[b].at[1]], ssems[b]).wait()

    def fire_idx(j, b):
        pltpu.async_copy(idx_hbm.at[wid, j], idxp[b], isems[b])

    def fire_gather(b):
        wait_idx(b)
        pltpu.async_copy(v_hbm.at[idxp[b].at[0]], rows[b], gsems[b])

    def fire_scatter(b):
        wait_gather(b)
        pltpu.async_copy(rows[b], acc.at[idxp[b].at[1]], ssems[b], add=True)

    for u in range(GRP):
        fire_idx(u, u)
        if u >= 2:
            fire_gather(u - 2)
        if u >= 4:
            fire_scatter(u - 4)

    def ring(t, _):
        for u in range(GRP):
            b = u
            wait_scatter(b)
            fire_idx(t * GRP + u, b)
            fire_gather((u - 2) % GRP)
            fire_scatter((u - 4) % GRP)
        return 0

    lax.fori_loop(1, NGRP, ring, 0)

    for u in range(TAIL):
        wait_scatter(u)
        fire_idx(NGRP * GRP + u, u)
        fire_gather((u - 2) % GRP)
        fire_scatter((u - 4) % GRP)
    for u in range(TAIL, TAIL + 2):
        fire_gather((u - 2) % GRP)
        fire_scatter((u - 4) % GRP)
    for u in range(TAIL + 2, TAIL + 4):
        fire_scatter((u - 4) % GRP)
    for b in range(GRP):
        wait_scatter(b)

    plsc.subcore_barrier()

    _foreach_tile_rows(s, lambda off, ln, k: pltpu.async_copy(
        acc.at[pl.ds(off, ln)], out_hbm.at[c, pl.ds(off, ln)], gsems[k]))


@functools.partial(
    pl.kernel,
    mesh=plsc.VectorSubcoreMesh(**_MESH),
    out_type=jax.ShapeDtypeStruct((NC, N, D), jnp.float32),
    scratch_types=[
        [pltpu.VMEM((CH,), jnp.int32)] * GRP,
        pltpu.VMEM((CH, D), jnp.float32),
        pltpu.VMEM_SHARED((NTRASH, D), jnp.float32),
        [pltpu.SemaphoreType.DMA] * GRP,
        [pltpu.SemaphoreType.DMA] * GRP,
    ],
)
def _sc_degree(idx_hbm, ones_hbm, zeros_hbm, out_hbm,
               sidx, ones_b, acc, isems, ssems):
    c = lax.axis_index("c")
    s = lax.axis_index("s")
    wid = s * NC + c

    pltpu.sync_copy(ones_hbm, ones_b)
    _foreach_tile_rows(s, lambda off, ln, k: pltpu.async_copy(
        zeros_hbm.at[pl.ds(0, ln)], acc.at[pl.ds(off, ln)], ssems[k]))

    @pl.when(s == 0)
    def _():
        pltpu.sync_copy(zeros_hbm.at[pl.ds(0, NTRASH - N)], acc.at[pl.ds(N, NTRASH - N)])

    plsc.subcore_barrier()

    def wait_scatter(b):
        pltpu.make_async_copy(ones_b, acc.at[sidx[b]], ssems[b]).wait()

    def ring(t, _):
        cps_i = []
        for b in range(GRP):
            @pl.when(t > 0)
            def _():
                wait_scatter(b)
            cps_i.append(pltpu.async_copy(idx_hbm.at[wid, t * GRP + b],
                                          sidx[b], isems[b]))
        for b in range(GRP):
            cps_i[b].wait()
            pltpu.async_copy(ones_b, acc.at[sidx[b]], ssems[b], add=True)
        return 0

    lax.fori_loop(0, NGRP, ring, 0)

    for b in range(TAIL):
        wait_scatter(b)
        pltpu.sync_copy(idx_hbm.at[wid, NGRP * GRP + b], sidx[b])
        pltpu.async_copy(ones_b, acc.at[sidx[b]], ssems[b], add=True)
    for b in range(GRP):
        wait_scatter(b)

    plsc.subcore_barrier()

    _foreach_tile_rows(s, lambda off, ln, k: pltpu.async_copy(
        acc.at[pl.ds(off, ln)], out_hbm.at[c, pl.ds(off, ln)], isems[k]))


def _dis_block(deg_ref):
    d0 = deg_ref[0, :, 0:1] + deg_ref[1, :, 0:1]
    return jnp.where(d0 > 0, lax.rsqrt(jnp.maximum(d0, 1.0)), 0.0)


_DEG_SPEC = pl.BlockSpec((NC, BN, D), lambda i: (0, i, 0))
_ROW_SPEC = pl.BlockSpec((BN, D), lambda i: (i, 0))
_S2_SPEC = pl.BlockSpec((NC, BN, D), lambda i: (0, i, 0))


def _tc_prescale(deg2, x):
    def body(deg_ref, x_ref, y_ref):
        y_ref[...] = x_ref[...] * _dis_block(deg_ref)

    return pl.pallas_call(
        body,
        grid=(N // BN,),
        in_specs=[_DEG_SPEC, _ROW_SPEC],
        out_specs=_ROW_SPEC,
        out_shape=jax.ShapeDtypeStruct((N, D), jnp.float32),
    )(deg2, x)


def _tc_scale(deg2, s2):
    def body(deg_ref, s_ref, t_ref, y_ref):
        dis = _dis_block(deg_ref)
        t = -dis * (s_ref[0] + s_ref[1])
        t_ref[...] = t
        y_ref[...] = t * dis

    return pl.pallas_call(
        body,
        grid=(N // BN,),
        in_specs=[_DEG_SPEC, _S2_SPEC],
        out_specs=[_ROW_SPEC, _ROW_SPEC],
        out_shape=[jax.ShapeDtypeStruct((N, D), jnp.float32),
                   jax.ShapeDtypeStruct((N, D), jnp.float32)],
    )(deg2, s2)


def _tc_cheb(deg2, s2, x0, t1, W, b):
    def body(deg_ref, s_ref, x0_ref, t1_ref, w_ref, b_ref, h_ref, y_ref):
        dis = _dis_block(deg_ref)
        x0b = x0_ref[...]
        tx2 = -2.0 * dis * (s_ref[0] + s_ref[1]) - x0b
        pre = (jnp.dot(x0b, w_ref[0], preferred_element_type=jnp.float32)
               + jnp.dot(t1_ref[...], w_ref[1], preferred_element_type=jnp.float32)
               + jnp.dot(tx2, w_ref[2], preferred_element_type=jnp.float32)
               + b_ref[...])
        h = jnp.maximum(pre, 0.0)
        h_ref[...] = h
        y_ref[...] = h * dis

    return pl.pallas_call(
        body,
        grid=(N // BN,),
        in_specs=[_DEG_SPEC, _S2_SPEC, _ROW_SPEC, _ROW_SPEC,
                  pl.BlockSpec((3, D, D), lambda i: (0, 0, 0)),
                  pl.BlockSpec((1, D), lambda i: (0, 0))],
        out_specs=[_ROW_SPEC, _ROW_SPEC],
        out_shape=[jax.ShapeDtypeStruct((N, D), jnp.float32),
                   jax.ShapeDtypeStruct((N, D), jnp.float32)],
    )(deg2, s2, x0, t1, W, b)


def _tc_cheb_pool(deg2, s2, x0, t1, W, b, batch3, Wlin, blin):
    nblk = N // BN

    def body(deg_ref, s_ref, x0_ref, t1_ref, w_ref, b_ref, batch_ref,
             wl_ref, bl_ref, out_ref, acc, cnt):
        i = pl.program_id(0)

        @pl.when(i == 0)
        def _():
            acc[...] = jnp.zeros_like(acc)
            cnt[...] = jnp.zeros_like(cnt)

        dis = _dis_block(deg_ref)
        x0b = x0_ref[...]
        tx2 = -2.0 * dis * (s_ref[0] + s_ref[1]) - x0b
        pre = (jnp.dot(x0b, w_ref[0], preferred_element_type=jnp.float32)
               + jnp.dot(t1_ref[...], w_ref[1], preferred_element_type=jnp.float32)
               + jnp.dot(tx2, w_ref[2], preferred_element_type=jnp.float32)
               + b_ref[...])
        g = jnp.maximum(pre, 0.0)

        bb = batch_ref[0, 0, :]
        oneh = (lax.broadcasted_iota(jnp.int32, (G, BN), 0)
                == bb[None, :]).astype(jnp.float32)
        acc[...] += jnp.dot(oneh, g, preferred_element_type=jnp.float32)
        cnt[...] += jnp.broadcast_to(jnp.sum(oneh, axis=1, keepdims=True), (G, D))

        @pl.when(i == nblk - 1)
        def _():
            pooled = acc[...] / jnp.maximum(cnt[...], 1.0)
            out_ref[...] = (jnp.dot(pooled, wl_ref[...],
                                    preferred_element_type=jnp.float32)
                            + bl_ref[...])

    return pl.pallas_call(
        body,
        grid=(nblk,),
        in_specs=[_DEG_SPEC, _S2_SPEC, _ROW_SPEC, _ROW_SPEC,
                  pl.BlockSpec((3, D, D), lambda i: (0, 0, 0)),
                  pl.BlockSpec((1, D), lambda i: (0, 0)),
                  pl.BlockSpec((1, 1, BN), lambda i: (i, 0, 0)),
                  pl.BlockSpec((D, DOUT), lambda i: (0, 0)),
                  pl.BlockSpec((1, DOUT), lambda i: (0, 0))],
        out_specs=pl.BlockSpec((G, DOUT), lambda i: (0, 0)),
        out_shape=jax.ShapeDtypeStruct((G, DOUT), jnp.float32),
        scratch_shapes=[pltpu.VMEM((G, D), jnp.float32),
                        pltpu.VMEM((G, D), jnp.float32)],
    )(deg2, s2, x0, t1, W, b, batch3, Wlin, blin)


def _tc_pool(g, batch3, Wlin, blin):
    nblk = N // BN

    def body(g_ref, b_ref, wl_ref, bl_ref, out_ref, acc, cnt):
        i = pl.program_id(0)

        @pl.when(i == 0)
        def _():
            acc[...] = jnp.zeros_like(acc)
            cnt[...] = jnp.zeros_like(cnt)

        bb = b_ref[0, 0, :]
        oneh = (lax.broadcasted_iota(jnp.int32, (G, BN), 0)
                == bb[None, :]).astype(jnp.float32)
        acc[...] += jnp.dot(oneh, g_ref[...], preferred_element_type=jnp.float32)
        cnt[...] += jnp.broadcast_to(jnp.sum(oneh, axis=1, keepdims=True), (G, D))

        @pl.when(i == nblk - 1)
        def _():
            pooled = acc[...] / jnp.maximum(cnt[...], 1.0)
            out_ref[...] = (jnp.dot(pooled, wl_ref[...],
                                    preferred_element_type=jnp.float32)
                            + bl_ref[...])

    return pl.pallas_call(
        body,
        grid=(nblk,),
        in_specs=[_ROW_SPEC,
                  pl.BlockSpec((1, 1, BN), lambda i: (i, 0, 0)),
                  pl.BlockSpec((D, DOUT), lambda i: (0, 0)),
                  pl.BlockSpec((1, DOUT), lambda i: (0, 0))],
        out_specs=pl.BlockSpec((G, DOUT), lambda i: (0, 0)),
        out_shape=jax.ShapeDtypeStruct((G, DOUT), jnp.float32),
        scratch_shapes=[pltpu.VMEM((G, D), jnp.float32),
                        pltpu.VMEM((G, D), jnp.float32)],
    )(g, batch3, Wlin, blin)


def kernel(x, edge_index, batch, W1, b1, W2, b2, Wlin, blin):
    row = edge_index[0].astype(jnp.int32).reshape(NW, EPW)
    col = edge_index[1].astype(jnp.int32).reshape(NW, EPW)
    pad_g = jnp.zeros((NW, NPADE), jnp.int32)
    pad_s = jnp.broadcast_to(N + (jnp.arange(NPADE, dtype=jnp.int32)
                                  % (NTRASH - N)), (NW, NPADE))
    rowg = jnp.concatenate([row, pad_g], axis=1).reshape(NW, CPW, CH)
    row_deg = jnp.concatenate([row, pad_s], axis=1).reshape(NW, CPW, CH)
    cols = jnp.concatenate([col, pad_s], axis=1).reshape(NW, CPW, CH)
    idx_e = jnp.stack([rowg, cols], axis=2)
    zeros_d = jnp.zeros((128, D), jnp.float32)
    ones_cd = jnp.ones((CH, D), jnp.float32)

    deg2 = _sc_degree(row_deg, ones_cd, zeros_d)
    y0 = _tc_prescale(deg2, x)

    s1 = _sc_spmm(y0, idx_e, zeros_d)
    tx1, y1 = _tc_scale(deg2, s1)
    s2 = _sc_spmm(y1, idx_e, zeros_d)
    h, y2 = _tc_cheb(deg2, s2, x, tx1, W1, b1.reshape(1, D))

    s3 = _sc_spmm(y2, idx_e, zeros_d)
    th1, y3 = _tc_scale(deg2, s3)
    s4 = _sc_spmm(y3, idx_e, zeros_d)
    batch3 = batch.astype(jnp.int32).reshape(N // BN, 1, BN)
    return _tc_cheb_pool(deg2, s4, h, th1, W2, b2.reshape(1, D),
                         batch3, Wlin, blin.reshape(1, DOUT))


# speedup vs baseline: 1.0144x; 1.0144x over previous
import functools

import jax
import jax.numpy as jnp
from jax import lax
from jax.experimental import pallas as pl
from jax.experimental.pallas import tpu as pltpu
from jax.experimental.pallas import tpu_sc as plsc

N = 10000
E = 320000
D = 128
DOUT = 64
G = 64
DEGW = 16
NC = 2
NS = 16
NW = NC * NS
CH = 64
GRP = 6
EPW = E // NW
CPW = 158
EW = CPW * CH
NPADE = EW - EPW
NGRP = CPW // GRP
TAIL = CPW - NGRP * GRP
NTRASH = N + 8
BN = 1000

RFULL = 632
_CHUNKS_HEAD = [(0, 128), (128, 128), (256, 128), (384, 128)]

_MESH = dict(core_axis_name="c", subcore_axis_name="s")


def _foreach_tile_rows(s, copy_fn):
    cps = []
    base = s * RFULL
    for k, (off, ln) in enumerate(_CHUNKS_HEAD):
        cps.append(copy_fn(pl.multiple_of(base + off, 8), ln, k))

    @pl.when(s < NS - 1)
    def _():
        copy_fn(pl.multiple_of(base + 512, 8), RFULL - 512, 4).wait()

    @pl.when(s == NS - 1)
    def _():
        copy_fn(pl.multiple_of(base + 512, 8), N - (NS - 1) * RFULL - 512, 4).wait()

    for cp in cps:
        cp.wait()


@functools.partial(
    pl.kernel,
    mesh=plsc.VectorSubcoreMesh(**_MESH),
    out_type=jax.ShapeDtypeStruct((NC, N, D), jnp.float32),
    scratch_types=[
        [pltpu.VMEM((2, CH), jnp.int32)] * GRP,
        [pltpu.VMEM((CH, D), jnp.float32)] * GRP,
        pltpu.VMEM_SHARED((NTRASH, D), jnp.float32),
        [pltpu.SemaphoreType.DMA] * GRP,
        [pltpu.SemaphoreType.DMA] * GRP,
        [pltpu.SemaphoreType.DMA] * GRP,
    ],
)
def _sc_spmm(v_hbm, idx_hbm, zeros_hbm, out_hbm,
             idxp, rows, acc, isems, gsems, ssems):
    c = lax.axis_index("c")
    s = lax.axis_index("s")
    wid = s * NC + c

    _foreach_tile_rows(s, lambda off, ln, k: pltpu.async_copy(
        zeros_hbm.at[pl.ds(0, ln)], acc.at[pl.ds(off, ln)], gsems[k]))

    @pl.when(s == 0)
    def _():
        pltpu.sync_copy(zeros_hbm.at[pl.ds(0, NTRASH - N)], acc.at[pl.ds(N, NTRASH - N)])

    plsc.subcore_barrier()

    def wait_idx(b):
        pltpu.make_async_copy(idx_hbm.at[wid, 0], idxp[b], isems[b]).wait()

    def wait_gather(b):
        pltpu.make_async_copy(v_hbm.at[idxp[b].at[0]], rows[b], gsems[b]).wait()

    def wait_scatter(b):
        pltpu.make_async_copy(rows[b], acc.at[idxp[b].at[1]], ssems[b]).wait()

    def fire_idx(j, b):
        pltpu.async_copy(idx_hbm.at[wid, j], idxp[b], isems[b])

    def fire_gather(b):
        wait_idx(b)
        pltpu.async_copy(v_hbm.at[idxp[b].at[0]], rows[b], gsems[b])

    def fire_scatter(b):
        wait_gather(b)
        pltpu.async_copy(rows[b], acc.at[idxp[b].at[1]], ssems[b], add=True)

    for u in range(GRP):
        fire_idx(u, u)
        if u >= 2:
            fire_gather(u - 2)
        if u >= 4:
            fire_scatter(u - 4)

    def ring(t, _):
        for u in range(GRP):
            b = u
            wait_scatter(b)
            fire_idx(t * GRP + u, b)
            fire_gather((u - 2) % GRP)
            fire_scatter((u - 4) % GRP)
        return 0

    lax.fori_loop(1, NGRP, ring, 0)

    for u in range(TAIL):
        wait_scatter(u)
        fire_idx(NGRP * GRP + u, u)
        fire_gather((u - 2) % GRP)
        fire_scatter((u - 4) % GRP)
    for u in range(TAIL, TAIL + 2):
        fire_gather((u - 2) % GRP)
        fire_scatter((u - 4) % GRP)
    for u in range(TAIL + 2, TAIL + 4):
        fire_scatter((u - 4) % GRP)
    for b in range(GRP):
        wait_scatter(b)

    plsc.subcore_barrier()

    _foreach_tile_rows(s, lambda off, ln, k: pltpu.async_copy(
        acc.at[pl.ds(off, ln)], out_hbm.at[c, pl.ds(off, ln)], gsems[k]))


@functools.partial(
    pl.kernel,
    mesh=plsc.VectorSubcoreMesh(**_MESH),
    out_type=jax.ShapeDtypeStruct((NC, N, D), jnp.float32),
    scratch_types=[
        [pltpu.VMEM((CH,), jnp.int32)] * GRP,
        pltpu.VMEM((CH, D), jnp.float32),
        pltpu.VMEM_SHARED((NTRASH, D), jnp.float32),
        [pltpu.SemaphoreType.DMA] * GRP,
        [pltpu.SemaphoreType.DMA] * GRP,
    ],
)
def _sc_degree(idx_hbm, ones_hbm, zeros_hbm, out_hbm,
               sidx, ones_b, acc, isems, ssems):
    c = lax.axis_index("c")
    s = lax.axis_index("s")
    wid = s * NC + c

    pltpu.sync_copy(ones_hbm, ones_b)
    _foreach_tile_rows(s, lambda off, ln, k: pltpu.async_copy(
        zeros_hbm.at[pl.ds(0, ln)], acc.at[pl.ds(off, ln)], ssems[k]))

    @pl.when(s == 0)
    def _():
        pltpu.sync_copy(zeros_hbm.at[pl.ds(0, NTRASH - N)], acc.at[pl.ds(N, NTRASH - N)])

    plsc.subcore_barrier()

    def wait_scatter(b):
        pltpu.make_async_copy(ones_b, acc.at[sidx[b]], ssems[b]).wait()

    def ring(t, _):
        cps_i = []
        for b in range(GRP):
            @pl.when(t > 0)
            def _():
                wait_scatter(b)
            cps_i.append(pltpu.async_copy(idx_hbm.at[wid, t * GRP + b],
                                          sidx[b], isems[b]))
        for b in range(GRP):
            cps_i[b].wait()
            pltpu.async_copy(ones_b, acc.at[sidx[b]], ssems[b], add=True)
        return 0

    lax.fori_loop(0, NGRP, ring, 0)

    for b in range(TAIL):
        wait_scatter(b)
        pltpu.sync_copy(idx_hbm.at[wid, NGRP * GRP + b], sidx[b])
        pltpu.async_copy(ones_b, acc.at[sidx[b]], ssems[b], add=True)
    for b in range(GRP):
        wait_scatter(b)

    plsc.subcore_barrier()

    _foreach_tile_rows(s, lambda off, ln, k: pltpu.async_copy(
        acc.at[pl.ds(off, ln)], out_hbm.at[c, pl.ds(off, ln)], isems[k]))


def _dis_block(deg_ref):
    d0 = deg_ref[0, :, 0:1] + deg_ref[1, :, 0:1]
    return jnp.where(d0 > 0, lax.rsqrt(jnp.maximum(d0, 1.0)), 0.0)


_DEG_SPEC = pl.BlockSpec((NC, BN, D), lambda i: (0, i, 0))
_ROW_SPEC = pl.BlockSpec((BN, D), lambda i: (i, 0))
_S2_SPEC = pl.BlockSpec((NC, BN, D), lambda i: (0, i, 0))


def _tc_prescale(deg2, x):
    def body(deg_ref, x_ref, y_ref):
        y_ref[...] = x_ref[...] * _dis_block(deg_ref)

    return pl.pallas_call(
        body,
        grid=(N // BN,),
        in_specs=[_DEG_SPEC, _ROW_SPEC],
        out_specs=_ROW_SPEC,
        out_shape=jax.ShapeDtypeStruct((N, D), jnp.float32),
    )(deg2, x)


def _tc_scale(deg2, s2):
    def body(deg_ref, s_ref, t_ref, y_ref):
        dis = _dis_block(deg_ref)
        t = -dis * (s_ref[0] + s_ref[1])
        t_ref[...] = t
        y_ref[...] = t * dis

    return pl.pallas_call(
        body,
        grid=(N // BN,),
        in_specs=[_DEG_SPEC, _S2_SPEC],
        out_specs=[_ROW_SPEC, _ROW_SPEC],
        out_shape=[jax.ShapeDtypeStruct((N, D), jnp.float32),
                   jax.ShapeDtypeStruct((N, D), jnp.float32)],
    )(deg2, s2)


def _tc_cheb(deg2, s2, x0, t1, W, b):
    def body(deg_ref, s_ref, x0_ref, t1_ref, w_ref, b_ref, h_ref, y_ref):
        dis = _dis_block(deg_ref)
        x0b = x0_ref[...]
        tx2 = -2.0 * dis * (s_ref[0] + s_ref[1]) - x0b
        pre = (jnp.dot(x0b, w_ref[0], preferred_element_type=jnp.float32)
               + jnp.dot(t1_ref[...], w_ref[1], preferred_element_type=jnp.float32)
               + jnp.dot(tx2, w_ref[2], preferred_element_type=jnp.float32)
               + b_ref[...])
        h = jnp.maximum(pre, 0.0)
        h_ref[...] = h
        y_ref[...] = h * dis

    return pl.pallas_call(
        body,
        grid=(N // BN,),
        in_specs=[_DEG_SPEC, _S2_SPEC, _ROW_SPEC, _ROW_SPEC,
                  pl.BlockSpec((3, D, D), lambda i: (0, 0, 0)),
                  pl.BlockSpec((1, D), lambda i: (0, 0))],
        out_specs=[_ROW_SPEC, _ROW_SPEC],
        out_shape=[jax.ShapeDtypeStruct((N, D), jnp.float32),
                   jax.ShapeDtypeStruct((N, D), jnp.float32)],
    )(deg2, s2, x0, t1, W, b)


def _tc_pool(g, batch3, Wlin, blin):
    nblk = N // BN

    def body(g_ref, b_ref, wl_ref, bl_ref, out_ref, acc, cnt):
        i = pl.program_id(0)

        @pl.when(i == 0)
        def _():
            acc[...] = jnp.zeros_like(acc)
            cnt[...] = jnp.zeros_like(cnt)

        bb = b_ref[0, 0, :]
        oneh = (lax.broadcasted_iota(jnp.int32, (G, BN), 0)
                == bb[None, :]).astype(jnp.float32)
        acc[...] += jnp.dot(oneh, g_ref[...], preferred_element_type=jnp.float32)
        cnt[...] += jnp.broadcast_to(jnp.sum(oneh, axis=1, keepdims=True), (G, D))

        @pl.when(i == nblk - 1)
        def _():
            pooled = acc[...] / jnp.maximum(cnt[...], 1.0)
            out_ref[...] = (jnp.dot(pooled, wl_ref[...],
                                    preferred_element_type=jnp.float32)
                            + bl_ref[...])

    return pl.pallas_call(
        body,
        grid=(nblk,),
        in_specs=[_ROW_SPEC,
                  pl.BlockSpec((1, 1, BN), lambda i: (i, 0, 0)),
                  pl.BlockSpec((D, DOUT), lambda i: (0, 0)),
                  pl.BlockSpec((1, DOUT), lambda i: (0, 0))],
        out_specs=pl.BlockSpec((G, DOUT), lambda i: (0, 0)),
        out_shape=jax.ShapeDtypeStruct((G, DOUT), jnp.float32),
        scratch_shapes=[pltpu.VMEM((G, D), jnp.float32),
                        pltpu.VMEM((G, D), jnp.float32)],
    )(g, batch3, Wlin, blin)


def kernel(x, edge_index, batch, W1, b1, W2, b2, Wlin, blin):
    row = edge_index[0].astype(jnp.int32).reshape(NW, EPW)
    col = edge_index[1].astype(jnp.int32).reshape(NW, EPW)
    pad_g = jnp.zeros((NW, NPADE), jnp.int32)
    pad_s = jnp.broadcast_to(N + (jnp.arange(NPADE, dtype=jnp.int32)
                                  % (NTRASH - N)), (NW, NPADE))
    rowg = jnp.concatenate([row, pad_g], axis=1).reshape(NW, CPW, CH)
    row_deg = jnp.concatenate([row, pad_s], axis=1).reshape(NW, CPW, CH)
    cols = jnp.concatenate([col, pad_s], axis=1).reshape(NW, CPW, CH)
    idx_e = jnp.stack([rowg, cols], axis=2)
    zeros_d = jnp.zeros((128, D), jnp.float32)
    ones_cd = jnp.ones((CH, D), jnp.float32)

    deg2 = _sc_degree(row_deg, ones_cd, zeros_d)
    y0 = _tc_prescale(deg2, x)

    s1 = _sc_spmm(y0, idx_e, zeros_d)
    tx1, y1 = _tc_scale(deg2, s1)
    s2 = _sc_spmm(y1, idx_e, zeros_d)
    h, y2 = _tc_cheb(deg2, s2, x, tx1, W1, b1.reshape(1, D))

    s3 = _sc_spmm(y2, idx_e, zeros_d)
    th1, y3 = _tc_scale(deg2, s3)
    s4 = _sc_spmm(y3, idx_e, zeros_d)
    g, _ = _tc_cheb(deg2, s4, h, th1, W2, b2.reshape(1, D))

    batch3 = batch.astype(jnp.int32).reshape(N // BN, 1, BN)
    return _tc_pool(g, batch3, Wlin, blin.reshape(1, DOUT))
